# baseline (speedup 1.0000x reference)
import functools

import jax
import jax.numpy as jnp
from jax import lax
from jax.experimental import pallas as pl
from jax.experimental.pallas import tpu as pltpu
from jax.experimental.pallas import tpu_sc as plsc

SEQ_LEN = 200
BATCH = 1024
D_MODEL = 64
DP = 128
NW = 32
TS = 8
TL = 128
NTL = BATCH // TL
NT = (SEQ_LEN // TS) * NTL
KMAX = -(-NT // NW)
NCH = KMAX * TS
NBUF = 7

_mesh = plsc.VectorSubcoreMesh(core_axis_name="c", subcore_axis_name="s")


@functools.partial(
    pl.kernel,
    mesh=_mesh,
    out_type=jax.ShapeDtypeStruct((SEQ_LEN, BATCH, DP), jnp.float32),
    scratch_types=[
        pltpu.VMEM((2, TS, TL), jnp.int32),
        pltpu.VMEM((NBUF, TL, DP), jnp.float32),
        pltpu.SemaphoreType.DMA,
    ],
)
def _gather(table_hbm, idx_hbm, out_hbm, idx_v, rows_v, gsem):
    wid = lax.axis_index("s") * 2 + lax.axis_index("c")

    def load_idx(k):
        t = wid + k * NW

        @pl.when(t < NT)
        def _():
            ts = t // NTL
            tl = t % NTL
            pltpu.sync_copy(
                idx_hbm.at[pl.ds(ts * TS, TS), pl.ds(tl * TL, TL)],
                idx_v.at[lax.rem(k, 2)],
            )

    def start(q):
        k = q // TS
        t = wid + k * NW

        @pl.when(t < NT)
        def _():
            pltpu.async_copy(
                table_hbm.at[idx_v.at[lax.rem(k, 2), lax.rem(q, TS)]],
                rows_v.at[lax.rem(q, NBUF)],
                gsem,
            )

    def finish(q):
        k = q // TS
        t = wid + k * NW

        @pl.when(t < NT)
        def _():
            ts = t // NTL
            tl = t % NTL
            pltpu.make_async_copy(
                table_hbm.at[idx_v.at[lax.rem(k, 2), lax.rem(q, TS)]],
                rows_v.at[lax.rem(q, NBUF)],
                gsem,
            ).wait()
            pltpu.sync_copy(
                rows_v.at[lax.rem(q, NBUF)],
                out_hbm.at[ts * TS + lax.rem(q, TS), pl.ds(tl * TL, TL)],
            )

    load_idx(0)
    for q in range(NBUF):
        start(q)

    def body(g, carry):
        nxt = g + NBUF

        @pl.when(lax.rem(nxt, TS) == 0)
        def _():
            load_idx(nxt // TS)

        finish(g)
        start(nxt)
        return carry

    lax.fori_loop(0, NCH - NBUF, body, 0)

    for q in range(NCH - NBUF, NCH):
        finish(q)


def kernel(input, table):
    table128 = jnp.concatenate([table, table], axis=1)
    out128 = _gather(table128, input)
    return out128[:, :, :D_MODEL]
